# baseline (speedup 1.0000x reference)
import functools

import jax
import jax.numpy as jnp
from jax import lax
from jax.experimental import pallas as pl
from jax.experimental.pallas import tpu as pltpu

_F8 = jnp.float8_e4m3fn


def _stage1_kernel(nu, n, br, nb, f_ref, w_ref, b2_ref, emb_ref, a_ref,
                   a8_ref, e1_ref, acc_ref, smax_ref, rs_ref, e0f, e08, sm):
    l = pl.program_id(0)
    r = pl.program_id(1)
    rows = pl.ds(r * br, br)
    gr = r * br + lax.broadcasted_iota(jnp.int32, (br, 1), 0)

    @pl.when(l == 0)
    def _prologue():
        feat = jnp.dot(f_ref[...], w_ref[...],
                       preferred_element_type=jnp.float32)
        bias = jnp.where(gr < nu, b2_ref[0:1, :], b2_ref[1:2, :])
        res = emb_ref[...] + feat + bias
        e0f[rows, :] = res
        e08[rows, :] = res.astype(jnp.bfloat16)

    @pl.when(l == 1)
    def _layer1():
        a32 = a_ref[...]
        a8_ref[...] = a32.astype(_F8)
        rs_ref[...] = jnp.broadcast_to(
            jnp.sum(a32, axis=1, keepdims=True), rs_ref.shape)
        res = jnp.dot(a32.astype(jnp.bfloat16), e08[pl.ds(0, n), :],
                      preferred_element_type=jnp.float32)
        e1_ref[...] = res.astype(jnp.bfloat16)
        acc_ref[...] = e0f[rows, :] + res
        m = jnp.max(jnp.where(gr < n, jnp.abs(res), 0.0), axis=0,
                    keepdims=True)
        prev = jnp.where(r == 0, jnp.zeros_like(m), sm[0:1, :])
        new = jnp.maximum(prev, m)
        sm[0:1, :] = new

        @pl.when(r == nb - 1)
        def _():
            smax_ref[...] = jnp.broadcast_to(new, smax_ref.shape)


def _stage2_kernel(n, br, a_ref, e1_ref, sm1_ref, acc01_ref, rs_ref, out_ref,
                   e18, e2f, e28, accs, sc, sm2):
    l = pl.program_id(0)
    r = pl.program_id(1)
    rows = pl.ds(r * br, br)
    gr = r * br + lax.broadcasted_iota(jnp.int32, (br, 1), 0)

    @pl.when(l == 0)
    def _layer2():
        @pl.when(r == 0)
        def _prep():
            cm = sm1_ref[0:1, :]
            s1 = jnp.where(cm > 0, cm, 192.0) * (1.0 / 192.0)
            sc[0:1, :] = s1
            e18[pl.ds(0, n), :] = (
                e1_ref[...].astype(jnp.float32) * (1.0 / s1)).astype(_F8)

        s1 = sc[0:1, :]
        res = jnp.dot(a_ref[...], e18[pl.ds(0, n), :],
                      preferred_element_type=jnp.float32)
        e2 = res * s1
        e2f[rows, :] = e2
        accs[rows, :] = acc01_ref[...] + e2
        m = jnp.sum(jnp.where(gr < n, e2, 0.0), axis=0, keepdims=True)
        prev = jnp.where(r == 0, jnp.zeros_like(m), sm2[0:1, :])
        sm2[0:1, :] = prev + m

    @pl.when(l == 1)
    def _layer3():
        @pl.when(r == 0)
        def _prep():
            mu = sm2[0:1, :] * (1.0 / n)
            sm2[0:1, :] = mu
            d = e2f[pl.ds(0, n), :] - mu
            cm = jnp.max(jnp.abs(d), axis=0, keepdims=True)
            s2 = jnp.where(cm > 0, cm, 192.0) * (1.0 / 192.0)
            sc[0:1, :] = s2
            e28[pl.ds(0, n), :] = (d * (1.0 / s2)).astype(_F8)

        s2 = sc[0:1, :]
        mu = sm2[0:1, :]
        res = jnp.dot(a_ref[...], e28[pl.ds(0, n), :],
                      preferred_element_type=jnp.float32)
        e3 = res * s2 + rs_ref[...][:, 0:1] * mu
        out_ref[...] = (accs[rows, :] + e3) * 0.25


@jax.jit
def kernel(adj, user_features, item_features, user_emb, item_emb, Wu, bu, Wi,
           bi):
    n = adj.shape[0]
    nu, fu = user_features.shape
    ni, fi = item_features.shape
    emb = user_emb.shape[1]
    br1 = 256
    nb1 = -(-n // br1)
    np1 = nb1 * br1
    br2 = 1248
    nb2 = -(-n // br2)
    np2 = nb2 * br2
    fk = fu + fi

    f = jnp.zeros((n, fk), jnp.float32)
    f = f.at[:nu, :fu].set(user_features)
    f = f.at[nu:, fu:].set(item_features)
    w = jnp.concatenate([Wu, Wi], axis=0)
    emb0 = jnp.concatenate([user_emb, item_emb], axis=0)
    b2 = jnp.zeros((8, emb), jnp.float32).at[0].set(bu).at[1].set(bi)

    body1 = functools.partial(_stage1_kernel, nu, n, br1, nb1)
    a8, e1, acc01, smax1, rs = pl.pallas_call(
        body1,
        grid=(2, nb1),
        in_specs=[
            pl.BlockSpec((br1, fk), lambda l, r: (jnp.where(l == 0, r, 0), 0)),
            pl.BlockSpec((fk, emb), lambda l, r: (0, 0)),
            pl.BlockSpec((8, emb), lambda l, r: (0, 0)),
            pl.BlockSpec((br1, emb), lambda l, r: (jnp.where(l == 0, r, 0), 0)),
            pl.BlockSpec((br1, n), lambda l, r: (jnp.where(l == 0, 0, r), 0)),
        ],
        out_specs=[
            pl.BlockSpec((br1, n), lambda l, r: (jnp.where(l == 1, r, 0), 0)),
            pl.BlockSpec((br1, emb), lambda l, r: (jnp.where(l == 1, r, 0), 0)),
            pl.BlockSpec((br1, emb), lambda l, r: (jnp.where(l == 1, r, 0), 0)),
            pl.BlockSpec((8, emb), lambda l, r: (0, 0)),
            pl.BlockSpec((br1, 8), lambda l, r: (jnp.where(l == 1, r, 0), 0)),
        ],
        out_shape=[
            jax.ShapeDtypeStruct((n, n), _F8),
            jax.ShapeDtypeStruct((n, emb), jnp.bfloat16),
            jax.ShapeDtypeStruct((n, emb), jnp.float32),
            jax.ShapeDtypeStruct((8, emb), jnp.float32),
            jax.ShapeDtypeStruct((n, 8), jnp.float32),
        ],
        scratch_shapes=[
            pltpu.VMEM((np1, emb), jnp.float32),
            pltpu.VMEM((np1, emb), jnp.bfloat16),
            pltpu.VMEM((8, emb), jnp.float32),
        ],
        compiler_params=pltpu.CompilerParams(
            dimension_semantics=("arbitrary", "arbitrary")),
    )(f, w, b2, emb0, adj)

    body2 = functools.partial(_stage2_kernel, n, br2)
    out = pl.pallas_call(
        body2,
        grid=(2, nb2),
        in_specs=[
            pl.BlockSpec((br2, n), lambda l, r: (r, 0)),
            pl.BlockSpec((n, emb), lambda l, r: (0, 0)),
            pl.BlockSpec((8, emb), lambda l, r: (0, 0)),
            pl.BlockSpec((br2, emb), lambda l, r: (jnp.where(l == 0, r, 0), 0)),
            pl.BlockSpec((br2, 8), lambda l, r: (jnp.where(l == 1, r, 0), 0)),
        ],
        out_specs=pl.BlockSpec((br2, emb), lambda l, r: (r, 0)),
        out_shape=jax.ShapeDtypeStruct((n, emb), jnp.float32),
        scratch_shapes=[
            pltpu.VMEM((np2, emb), _F8),
            pltpu.VMEM((np2, emb), jnp.float32),
            pltpu.VMEM((np2, emb), _F8),
            pltpu.VMEM((np2, emb), jnp.float32),
            pltpu.VMEM((8, emb), jnp.float32),
            pltpu.VMEM((8, emb), jnp.float32),
        ],
        compiler_params=pltpu.CompilerParams(
            dimension_semantics=("arbitrary", "arbitrary")),
    )(a8, e1, smax1, acc01, rs)

    return out[:nu], out[nu:]


# speedup vs baseline: 1.0389x; 1.0389x over previous
import functools

import jax
import jax.numpy as jnp
from jax import lax
from jax.experimental import pallas as pl
from jax.experimental.pallas import tpu as pltpu

_F8 = jnp.float8_e4m3fn


def _stage1_kernel(nu, n, br, nb, f_ref, w_ref, b2_ref, emb_ref, a_ref,
                   a8_ref, e1_ref, acc_ref, smax_ref, rs_ref, e08, sm):
    l = pl.program_id(0)
    r = pl.program_id(1)
    rows = pl.ds(r * br, br)
    gr = r * br + lax.broadcasted_iota(jnp.int32, (br, 1), 0)

    @pl.when(l == 0)
    def _prologue():
        feat = jnp.dot(f_ref[...], w_ref[...],
                       preferred_element_type=jnp.float32)
        bias = jnp.where(gr < nu, b2_ref[0:1, :], b2_ref[1:2, :])
        res = emb_ref[...] + feat + bias
        e08[rows, :] = res.astype(jnp.bfloat16)

    @pl.when(l == 1)
    def _layer1():
        a32 = a_ref[...]
        a8_ref[...] = a32.astype(_F8)
        rs_ref[...] = jnp.broadcast_to(
            jnp.sum(a32, axis=1, keepdims=True), rs_ref.shape)
        res = jnp.dot(a32.astype(jnp.bfloat16), e08[pl.ds(0, n), :],
                      preferred_element_type=jnp.float32)
        e1_ref[...] = res.astype(jnp.bfloat16)
        acc_ref[...] = e08[rows, :].astype(jnp.float32) + res
        m = jnp.max(jnp.where(gr < n, jnp.abs(res), 0.0), axis=0,
                    keepdims=True)
        prev = jnp.where(r == 0, jnp.zeros_like(m), sm[0:1, :])
        new = jnp.maximum(prev, m)
        sm[0:1, :] = new

        @pl.when(r == nb - 1)
        def _():
            smax_ref[...] = jnp.broadcast_to(new, smax_ref.shape)


def _stage2_kernel(n, br, a_ref, e1_ref, sm1_ref, acc01_ref, rs_ref, out_ref,
                   e18, e2f, e28, accs, sc, sm2):
    l = pl.program_id(0)
    r = pl.program_id(1)
    rows = pl.ds(r * br, br)
    gr = r * br + lax.broadcasted_iota(jnp.int32, (br, 1), 0)

    @pl.when(l == 0)
    def _layer2():
        @pl.when(r == 0)
        def _prep():
            cm = sm1_ref[0:1, :]
            s1 = jnp.where(cm > 0, cm, 192.0) * (1.0 / 192.0)
            sc[0:1, :] = s1
            e18[pl.ds(0, n), :] = (
                e1_ref[...].astype(jnp.float32) * (1.0 / s1)).astype(_F8)

        s1 = sc[0:1, :]
        res = jnp.dot(a_ref[...], e18[pl.ds(0, n), :],
                      preferred_element_type=jnp.float32)
        e2 = res * s1
        e2f[rows, :] = e2
        accs[rows, :] = acc01_ref[...] + e2
        m = jnp.sum(jnp.where(gr < n, e2, 0.0), axis=0, keepdims=True)
        prev = jnp.where(r == 0, jnp.zeros_like(m), sm2[0:1, :])
        sm2[0:1, :] = prev + m

    @pl.when(l == 1)
    def _layer3():
        @pl.when(r == 0)
        def _prep():
            mu = sm2[0:1, :] * (1.0 / n)
            sm2[0:1, :] = mu
            d = e2f[pl.ds(0, n), :] - mu
            cm = jnp.max(jnp.abs(d), axis=0, keepdims=True)
            s2 = jnp.where(cm > 0, cm, 192.0) * (1.0 / 192.0)
            sc[0:1, :] = s2
            e28[pl.ds(0, n), :] = (d * (1.0 / s2)).astype(_F8)

        s2 = sc[0:1, :]
        mu = sm2[0:1, :]
        res = jnp.dot(a_ref[...], e28[pl.ds(0, n), :],
                      preferred_element_type=jnp.float32)
        e3 = res * s2 + rs_ref[...][:, 0:1] * mu
        out_ref[...] = (accs[rows, :] + e3) * 0.25


@jax.jit
def kernel(adj, user_features, item_features, user_emb, item_emb, Wu, bu, Wi,
           bi):
    n = adj.shape[0]
    nu, fu = user_features.shape
    ni, fi = item_features.shape
    emb = user_emb.shape[1]
    br1 = 512
    nb1 = -(-n // br1)
    np1 = nb1 * br1
    br2 = 1248
    nb2 = -(-n // br2)
    np2 = nb2 * br2
    fk = fu + fi

    f = jnp.zeros((n, fk), jnp.float32)
    f = f.at[:nu, :fu].set(user_features)
    f = f.at[nu:, fu:].set(item_features)
    w = jnp.concatenate([Wu, Wi], axis=0)
    emb0 = jnp.concatenate([user_emb, item_emb], axis=0)
    b2 = jnp.zeros((8, emb), jnp.float32).at[0].set(bu).at[1].set(bi)

    body1 = functools.partial(_stage1_kernel, nu, n, br1, nb1)
    a8, e1, acc01, smax1, rs = pl.pallas_call(
        body1,
        grid=(2, nb1),
        in_specs=[
            pl.BlockSpec((br1, fk), lambda l, r: (jnp.where(l == 0, r, 0), 0)),
            pl.BlockSpec((fk, emb), lambda l, r: (0, 0)),
            pl.BlockSpec((8, emb), lambda l, r: (0, 0)),
            pl.BlockSpec((br1, emb), lambda l, r: (jnp.where(l == 0, r, 0), 0)),
            pl.BlockSpec((br1, n), lambda l, r: (jnp.where(l == 0, 0, r), 0)),
        ],
        out_specs=[
            pl.BlockSpec((br1, n), lambda l, r: (jnp.where(l == 1, r, 0), 0)),
            pl.BlockSpec((br1, emb), lambda l, r: (jnp.where(l == 1, r, 0), 0)),
            pl.BlockSpec((br1, emb), lambda l, r: (jnp.where(l == 1, r, 0), 0)),
            pl.BlockSpec((8, emb), lambda l, r: (0, 0)),
            pl.BlockSpec((br1, 8), lambda l, r: (jnp.where(l == 1, r, 0), 0)),
        ],
        out_shape=[
            jax.ShapeDtypeStruct((n, n), _F8),
            jax.ShapeDtypeStruct((n, emb), jnp.bfloat16),
            jax.ShapeDtypeStruct((n, emb), jnp.float32),
            jax.ShapeDtypeStruct((8, emb), jnp.float32),
            jax.ShapeDtypeStruct((n, 8), jnp.float32),
        ],
        scratch_shapes=[
            pltpu.VMEM((np1, emb), jnp.bfloat16),
            pltpu.VMEM((8, emb), jnp.float32),
        ],
        compiler_params=pltpu.CompilerParams(
            dimension_semantics=("arbitrary", "arbitrary")),
    )(f, w, b2, emb0, adj)

    body2 = functools.partial(_stage2_kernel, n, br2)
    out = pl.pallas_call(
        body2,
        grid=(2, nb2),
        in_specs=[
            pl.BlockSpec((br2, n), lambda l, r: (r, 0)),
            pl.BlockSpec((n, emb), lambda l, r: (0, 0)),
            pl.BlockSpec((8, emb), lambda l, r: (0, 0)),
            pl.BlockSpec((br2, emb), lambda l, r: (jnp.where(l == 0, r, 0), 0)),
            pl.BlockSpec((br2, 8), lambda l, r: (jnp.where(l == 1, r, 0), 0)),
        ],
        out_specs=pl.BlockSpec((br2, emb), lambda l, r: (r, 0)),
        out_shape=jax.ShapeDtypeStruct((n, emb), jnp.float32),
        scratch_shapes=[
            pltpu.VMEM((np2, emb), _F8),
            pltpu.VMEM((np2, emb), jnp.float32),
            pltpu.VMEM((np2, emb), _F8),
            pltpu.VMEM((np2, emb), jnp.float32),
            pltpu.VMEM((8, emb), jnp.float32),
            pltpu.VMEM((8, emb), jnp.float32),
        ],
        compiler_params=pltpu.CompilerParams(
            dimension_semantics=("arbitrary", "arbitrary")),
    )(a8, e1, smax1, acc01, rs)

    return out[:nu], out[nu:]
